# baseline (speedup 1.0000x reference)
import jax
import jax.numpy as jnp
from jax.experimental import pallas as pl

N = 10000
G = 64


def _gcn_conv(x, src, dst, ew, W, b, n):
    loop = jnp.arange(n)
    s = jnp.concatenate([src, loop])
    d = jnp.concatenate([dst, loop])
    w = jnp.concatenate([ew, jnp.ones((n,), dtype=x.dtype)])
    deg = jnp.zeros((n,), dtype=x.dtype).at[d].add(w)
    dinv = jnp.where(deg > 0, 1.0 / jnp.sqrt(deg), 0.0)
    norm = dinv[s] * w * dinv[d]
    h = x @ W
    msg = h[s] * norm[:, None]
    out = jnp.zeros((n, W.shape[1]), dtype=x.dtype).at[d].add(msg)
    return out + b


def _div_kernel(s_ref, c_ref, o_ref):
    o_ref[...] = s_ref[...] / jnp.maximum(c_ref[...], 1.0)


def kernel(x, edge_index, edge_attr, batch, W0, b0, W1, b1, W2, b2, R0, rb0, R1, rb1):
    src = edge_index[0]
    dst = edge_index[1]
    h = _gcn_conv(x, src, dst, edge_attr, W0, b0, N)
    h = jax.nn.relu(h)
    h = _gcn_conv(h, src, dst, edge_attr, W1, b1, N)
    h = jax.nn.relu(h)
    h = _gcn_conv(h, src, dst, edge_attr, W2, b2, N)
    r = jax.nn.relu(h @ R0 + rb0) @ R1 + rb1
    sums = jax.ops.segment_sum(r, batch, num_segments=G)
    cnts = jax.ops.segment_sum(jnp.ones((N, 1), dtype=r.dtype), batch, num_segments=G)
    out = pl.pallas_call(
        _div_kernel,
        out_shape=jax.ShapeDtypeStruct((G, 1), jnp.float32),
    )(sums, cnts)
    return out


# speedup vs baseline: 6.3786x; 6.3786x over previous
import jax
import jax.numpy as jnp
from jax import lax
from jax.experimental import pallas as pl
from jax.experimental.pallas import tpu as pltpu
from jax.experimental.pallas import tpu_sc as plsc

N = 10000
E = 320000
D = 128
G = 64

NC, NS = 2, 16
NW = NC * NS
NP = 10240
RPT = NP // NS
CH, C = 80, 128
EW = CH * C
EP = NW * EW

_f32 = jnp.float32
_i32 = jnp.int32


def _mesh():
    return plsc.VectorSubcoreMesh(core_axis_name="c", subcore_axis_name="s",
                                  num_cores=NC, num_subcores=NS)


def _zero16(ref, ngroups):
    def _z(g, carry):
        ref[g // 8, pl.ds((g % 8) * 16, 16)] = jnp.zeros((16,), _f32)
        return carry
    lax.fori_loop(0, ngroups, _z, None)


def _pre_body(dsts, ews, srcs, dinv_out, coef_out,
              dstv, ewv, srcv, degp, sumb, dinvc, dinv_full,
              coefv, sh_deg, sh_dinv):
    c = lax.axis_index("c")
    s = lax.axis_index("s")

    def _zd(i, carry):
        degp[pl.ds(i * 16, 16)] = jnp.zeros((16,), _f32)
        return carry
    lax.fori_loop(0, NP // 16, _zd, None)

    def _slice(k):
        pltpu.sync_copy(dsts.at[k], dstv)
        pltpu.sync_copy(ews.at[k], ewv)
        def _g(g, carry):
            r, col = g // 8, (g % 8) * 16
            d16 = dstv[r, pl.ds(col, 16)]
            w16 = ewv[r, pl.ds(col, 16)]
            plsc.addupdate_scatter(degp, [d16], w16)
            return carry
        lax.fori_loop(0, CH * 8, _g, None)
    _slice(2 * s)
    _slice(2 * s + 1)

    pltpu.sync_copy(degp, sh_deg.at[s])
    plsc.subcore_barrier()

    pltpu.sync_copy(sh_deg.at[:, pl.ds(s * RPT, RPT)], sumb)
    def _rs(v, carry):
        dsum = jnp.zeros((16,), _f32)
        for r in range(NS):
            dsum = dsum + sumb[r, pl.ds(v * 16, 16)]
        dsum = dsum + 1.0
        i = plsc.bitcast(dsum, _i32)
        i = 0x5F3759DF - lax.shift_right_logical(i, 1)
        y = plsc.bitcast(i, _f32)
        for _ in range(3):
            y = y * (1.5 - 0.5 * dsum * y * y)
        dinvc[pl.ds(v * 16, 16)] = y
        return carry
    lax.fori_loop(0, 40, _rs, None)
    pltpu.sync_copy(dinvc, sh_dinv.at[pl.ds(s * RPT, RPT)])

    @pl.when(c == 0)
    def _():
        pltpu.sync_copy(dinvc, dinv_out.at[pl.ds(s * RPT, RPT)])
    plsc.subcore_barrier()

    pltpu.sync_copy(sh_dinv, dinv_full)
    wid = c * NS + s
    pltpu.sync_copy(srcs.at[wid], srcv)
    pltpu.sync_copy(dsts.at[wid], dstv)
    pltpu.sync_copy(ews.at[wid], ewv)
    def _ce(g, carry):
        r, col = g // 8, (g % 8) * 16
        s16 = srcv[r, pl.ds(col, 16)]
        d16 = dstv[r, pl.ds(col, 16)]
        w16 = ewv[r, pl.ds(col, 16)]
        cc = w16 * plsc.load_gather(dinv_full, [s16]) * plsc.load_gather(dinv_full, [d16])
        coefv[r, pl.ds(col, 16)] = cc
        return carry
    lax.fori_loop(0, CH * 8, _ce, None)
    pltpu.sync_copy(coefv, coef_out.at[wid])


def _pre(dsts, ews, srcs):
    f = pl.kernel(
        _pre_body,
        out_type=(jax.ShapeDtypeStruct((NP,), _f32),
                  jax.ShapeDtypeStruct((NW, CH, C), _f32)),
        mesh=_mesh(),
        compiler_params=pltpu.CompilerParams(needs_layout_passes=False),
        scratch_types=[
            pltpu.VMEM((CH, C), _i32),
            pltpu.VMEM((CH, C), _f32),
            pltpu.VMEM((CH, C), _i32),
            pltpu.VMEM((NP,), _f32),
            pltpu.VMEM((NS, RPT), _f32),
            pltpu.VMEM((RPT,), _f32),
            pltpu.VMEM((NP,), _f32),
            pltpu.VMEM((CH, C), _f32),
            pltpu.VMEM_SHARED((NS, NP), _f32),
            pltpu.VMEM_SHARED((NP,), _f32),
        ],
    )
    return f(dsts, ews, srcs)


def _prop_body(h, srcs, dsts, coefs, acc_out,
               srcv, dstv, coefv, rows, acc_sh, sem):
    c = lax.axis_index("c")
    s = lax.axis_index("s")
    wid = c * NS + s

    _zero16(rows, C * 8)
    for k in range(RPT // C):
        pltpu.sync_copy(rows, acc_sh.at[pl.ds(s * RPT + k * C, C), :])
    plsc.subcore_barrier()

    pltpu.sync_copy(srcs.at[wid], srcv)
    pltpu.sync_copy(dsts.at[wid], dstv)
    pltpu.sync_copy(coefs.at[wid], coefv)

    def _chunk(j, carry):
        pltpu.async_copy(h.at[srcv.at[j]], rows, sem).wait()
        def _grp(g8, c2):
            c16 = coefv[j, pl.ds(g8 * 16, 16)]
            for r16 in range(16):
                e = g8 * 16 + r16
                bc = c16.at[jnp.full((16,), r16, _i32)].get(
                    mode="promise_in_bounds")
                for k in range(8):
                    sl = pl.ds(k * 16, 16)
                    rows[e, sl] = rows[e, sl] * bc
            return c2
        lax.fori_loop(0, 8, _grp, None)
        pltpu.sync_copy(rows, acc_sh.at[dstv.at[j]], add=True)
        return carry
    lax.fori_loop(0, CH, _chunk, None)
    plsc.subcore_barrier()

    pltpu.sync_copy(acc_sh.at[pl.ds(s * RPT, RPT), :],
                    acc_out.at[c, pl.ds(s * RPT, RPT), :])


def _prop(h, srcs, dsts, coefs):
    f = pl.kernel(
        _prop_body,
        out_type=jax.ShapeDtypeStruct((NC, NP, D), _f32),
        mesh=_mesh(),
        compiler_params=pltpu.CompilerParams(needs_layout_passes=False),
        scratch_types=[
            pltpu.VMEM((CH, C), _i32),
            pltpu.VMEM((CH, C), _i32),
            pltpu.VMEM((CH, C), _f32),
            pltpu.VMEM((C, D), _f32),
            pltpu.VMEM_SHARED((NP, D), _f32),
            pltpu.SemaphoreType.DMA,
        ],
    )
    return f(h, srcs, dsts, coefs)


_BM = 512


def _mm0_body(x_ref, w_ref, o_ref):
    o_ref[...] = jnp.dot(x_ref[...], w_ref[...], preferred_element_type=_f32)


def _mm0(xp, W):
    return pl.pallas_call(
        _mm0_body,
        grid=(NP // _BM,),
        in_specs=[pl.BlockSpec((_BM, D), lambda i: (i, 0)),
                  pl.BlockSpec((D, D), lambda i: (0, 0))],
        out_specs=pl.BlockSpec((_BM, D), lambda i: (i, 0)),
        out_shape=jax.ShapeDtypeStruct((NP, D), _f32),
    )(xp, W)


def _mid_body(a0_ref, a1_ref, h_ref, di_ref, b_ref, w_ref, o_ref):
    d = di_ref[...]
    xb = a0_ref[...] + a1_ref[...] + d * d * h_ref[...] + b_ref[...]
    xb = jnp.maximum(xb, 0.0)
    o_ref[...] = jnp.dot(xb, w_ref[...], preferred_element_type=_f32)


def _mid(a0, a1, h, dinv2d, br, W):
    return pl.pallas_call(
        _mid_body,
        grid=(NP // _BM,),
        in_specs=[pl.BlockSpec((_BM, D), lambda i: (i, 0)),
                  pl.BlockSpec((_BM, D), lambda i: (i, 0)),
                  pl.BlockSpec((_BM, D), lambda i: (i, 0)),
                  pl.BlockSpec((_BM, 1), lambda i: (i, 0)),
                  pl.BlockSpec((1, D), lambda i: (0, 0)),
                  pl.BlockSpec((D, D), lambda i: (0, 0))],
        out_specs=pl.BlockSpec((_BM, D), lambda i: (i, 0)),
        out_shape=jax.ShapeDtypeStruct((NP, D), _f32),
    )(a0, a1, h, dinv2d, br, W)


_BM7 = 256


def _read_body(a0_ref, a1_ref, h_ref, di_ref, b_ref, r0_ref, rb0_ref, r1_ref,
               rb1_ref, bt_ref, o_ref, acc_s, acc_c):
    i = pl.program_id(0)

    @pl.when(i == 0)
    def _():
        acc_s[...] = jnp.zeros_like(acc_s)
        acc_c[...] = jnp.zeros_like(acc_c)

    d = di_ref[...]
    x2 = a0_ref[...] + a1_ref[...] + d * d * h_ref[...] + b_ref[...]
    t = jnp.maximum(
        jnp.dot(x2, r0_ref[...], preferred_element_type=_f32) + rb0_ref[...], 0.0)
    r = jnp.dot(t, r1_ref[...], preferred_element_type=_f32) + rb1_ref[0, 0]
    oh = (bt_ref[...] == lax.broadcasted_iota(_i32, (_BM7, 128), 1)).astype(_f32)
    acc_s[...] += lax.dot_general(oh, r, (((0,), (0,)), ((), ())),
                                  preferred_element_type=_f32)
    acc_c[...] += lax.dot_general(oh, jnp.ones((_BM7, 1), _f32),
                                  (((0,), (0,)), ((), ())),
                                  preferred_element_type=_f32)

    @pl.when(i == pl.num_programs(0) - 1)
    def _():
        o_ref[...] = acc_s[...] / jnp.maximum(acc_c[...], 1.0)


def _read(a0, a1, h, dinv2d, br, R0p, rb0p, R1p, rb1p, bt):
    return pl.pallas_call(
        _read_body,
        grid=(NP // _BM7,),
        in_specs=[pl.BlockSpec((_BM7, D), lambda i: (i, 0)),
                  pl.BlockSpec((_BM7, D), lambda i: (i, 0)),
                  pl.BlockSpec((_BM7, D), lambda i: (i, 0)),
                  pl.BlockSpec((_BM7, 1), lambda i: (i, 0)),
                  pl.BlockSpec((1, D), lambda i: (0, 0)),
                  pl.BlockSpec((D, D), lambda i: (0, 0)),
                  pl.BlockSpec((1, D), lambda i: (0, 0)),
                  pl.BlockSpec((D, 1), lambda i: (0, 0)),
                  pl.BlockSpec((1, 1), lambda i: (0, 0)),
                  pl.BlockSpec((_BM7, 1), lambda i: (i, 0))],
        out_specs=pl.BlockSpec((128, 1), lambda i: (0, 0)),
        out_shape=jax.ShapeDtypeStruct((128, 1), _f32),
        scratch_shapes=[pltpu.VMEM((128, 1), _f32),
                        pltpu.VMEM((128, 1), _f32)],
    )(a0, a1, h, dinv2d, br, R0p, rb0p, R1p, rb1p, bt)


def kernel(x, edge_index, edge_attr, batch, W0, b0, W1, b1, W2, b2, R0, rb0, R1, rb1):
    src = edge_index[0]
    dst = edge_index[1]
    srcs = jnp.pad(src, (0, EP - E)).reshape(NW, CH, C)
    dsts = jnp.pad(dst, (0, EP - E)).reshape(NW, CH, C)
    ews = jnp.pad(edge_attr, (0, EP - E)).reshape(NW, CH, C)
    xp = jnp.pad(x, ((0, NP - N), (0, 0)))
    bt = jnp.pad(batch, (0, NP - N), constant_values=127).reshape(NP, 1)
    R0p = jnp.pad(R0, ((0, 0), (0, 128 - R0.shape[1])))
    rb0p = jnp.pad(rb0, (0, 128 - rb0.shape[0])).reshape(1, 128)
    R1p = jnp.pad(R1, ((0, 128 - R1.shape[0]), (0, 0)))
    rb1p = rb1.reshape(1, 1)
    b0r = b0.reshape(1, D)
    b1r = b1.reshape(1, D)
    b2r = b2.reshape(1, D)

    dinv, coef = _pre(dsts, ews, srcs)
    dinv2d = dinv.reshape(NP, 1)

    h0 = _mm0(xp, W0)
    acc = _prop(h0, srcs, dsts, coef)
    h1 = _mid(acc[0], acc[1], h0, dinv2d, b0r, W1)
    acc = _prop(h1, srcs, dsts, coef)
    h2 = _mid(acc[0], acc[1], h1, dinv2d, b1r, W2)
    acc = _prop(h2, srcs, dsts, coef)
    outp = _read(acc[0], acc[1], h2, dinv2d, b2r, R0p, rb0p, R1p, rb1p, bt)
    return outp[:G]


# speedup vs baseline: 7.0689x; 1.1082x over previous
import jax
import jax.numpy as jnp
from jax import lax
from jax.experimental import pallas as pl
from jax.experimental.pallas import tpu as pltpu
from jax.experimental.pallas import tpu_sc as plsc

N = 10000
E = 320000
D = 128
G = 64

NC, NS = 2, 16
NW = NC * NS
NP = 10240
RPT = NP // NS
CH, C = 80, 128
QC = 16
EW = CH * C
EP = NW * EW

_f32 = jnp.float32
_i32 = jnp.int32


def _mesh():
    return plsc.VectorSubcoreMesh(core_axis_name="c", subcore_axis_name="s",
                                  num_cores=NC, num_subcores=NS)


def _zero16(ref, ngroups):
    def _z(g, carry):
        ref[g // 8, pl.ds((g % 8) * 16, 16)] = jnp.zeros((16,), _f32)
        return carry
    lax.fori_loop(0, ngroups, _z, None)


def _pre_body(dsts, ews, srcs, dinv_out, coef_out,
              dstv, ewv, srcv, degp, sumb, dinvc, dinv_full,
              coefv, sh_deg, sh_dinv):
    c = lax.axis_index("c")
    s = lax.axis_index("s")

    def _zd(i, carry):
        degp[pl.ds(i * 16, 16)] = jnp.zeros((16,), _f32)
        return carry
    lax.fori_loop(0, NP // 16, _zd, None)

    def _slice(k):
        pltpu.sync_copy(dsts.at[k], dstv)
        pltpu.sync_copy(ews.at[k], ewv)
        def _g(g, carry):
            r, col = g // 8, (g % 8) * 16
            d16 = dstv[r, pl.ds(col, 16)]
            w16 = ewv[r, pl.ds(col, 16)]
            plsc.addupdate_scatter(degp, [d16], w16)
            return carry
        lax.fori_loop(0, CH * 8, _g, None)
    _slice(2 * s)
    _slice(2 * s + 1)

    pltpu.sync_copy(degp, sh_deg.at[s])
    plsc.subcore_barrier()

    pltpu.sync_copy(sh_deg.at[:, pl.ds(s * RPT, RPT)], sumb)
    def _rs(v, carry):
        dsum = jnp.zeros((16,), _f32)
        for r in range(NS):
            dsum = dsum + sumb[r, pl.ds(v * 16, 16)]
        dsum = dsum + 1.0
        i = plsc.bitcast(dsum, _i32)
        i = 0x5F3759DF - lax.shift_right_logical(i, 1)
        y = plsc.bitcast(i, _f32)
        for _ in range(3):
            y = y * (1.5 - 0.5 * dsum * y * y)
        dinvc[pl.ds(v * 16, 16)] = y
        return carry
    lax.fori_loop(0, 40, _rs, None)
    pltpu.sync_copy(dinvc, sh_dinv.at[pl.ds(s * RPT, RPT)])

    @pl.when(c == 0)
    def _():
        pltpu.sync_copy(dinvc, dinv_out.at[pl.ds(s * RPT, RPT)])
    plsc.subcore_barrier()

    pltpu.sync_copy(sh_dinv, dinv_full)
    wid = c * NS + s
    pltpu.sync_copy(srcs.at[wid], srcv)
    pltpu.sync_copy(dsts.at[wid], dstv)
    pltpu.sync_copy(ews.at[wid], ewv)
    def _ce(g, carry):
        r, col = g // 8, (g % 8) * 16
        s16 = srcv[r, pl.ds(col, 16)]
        d16 = dstv[r, pl.ds(col, 16)]
        w16 = ewv[r, pl.ds(col, 16)]
        cc = w16 * plsc.load_gather(dinv_full, [s16]) * plsc.load_gather(dinv_full, [d16])
        coefv[r, pl.ds(col, 16)] = cc
        return carry
    lax.fori_loop(0, CH * 8, _ce, None)
    pltpu.sync_copy(coefv, coef_out.at[wid])


def _pre(dsts, ews, srcs):
    f = pl.kernel(
        _pre_body,
        out_type=(jax.ShapeDtypeStruct((NP,), _f32),
                  jax.ShapeDtypeStruct((NW, CH, C), _f32)),
        mesh=_mesh(),
        compiler_params=pltpu.CompilerParams(needs_layout_passes=False),
        scratch_types=[
            pltpu.VMEM((CH, C), _i32),
            pltpu.VMEM((CH, C), _f32),
            pltpu.VMEM((CH, C), _i32),
            pltpu.VMEM((NP,), _f32),
            pltpu.VMEM((NS, RPT), _f32),
            pltpu.VMEM((RPT,), _f32),
            pltpu.VMEM((NP,), _f32),
            pltpu.VMEM((CH, C), _f32),
            pltpu.VMEM_SHARED((NS, NP), _f32),
            pltpu.VMEM_SHARED((NP,), _f32),
        ],
    )
    return f(dsts, ews, srcs)


def _prop_body(h, srcs, dsts, coefs, acc_out,
               srcv, dstv, coefv, rows0, rows1, acc_sh, sem0, sem1):
    c = lax.axis_index("c")
    s = lax.axis_index("s")
    wid = c * NS + s

    _zero16(rows0, C * 8)
    for k in range(RPT // C):
        pltpu.sync_copy(rows0, acc_sh.at[pl.ds(s * RPT + k * C, C), :])
    plsc.subcore_barrier()

    def _scale(rows, j):
        def _grp(g8, c2):
            c16 = coefv[j, pl.ds(g8 * 16, 16)]
            for r16 in range(16):
                e = g8 * 16 + r16
                bc = c16.at[jnp.full((16,), r16, _i32)].get(
                    mode="promise_in_bounds")
                for k in range(8):
                    sl = pl.ds(k * 16, 16)
                    rows[e, sl] = rows[e, sl] * bc
            return c2
        lax.fori_loop(0, 8, _grp, None)

    def _quarter(q, carry):
        off = pl.multiple_of(q * QC, 8)
        pltpu.sync_copy(srcs.at[wid, pl.ds(off, QC), :], srcv)
        pltpu.sync_copy(dsts.at[wid, pl.ds(off, QC), :], dstv)
        pltpu.sync_copy(coefs.at[wid, pl.ds(off, QC), :], coefv)
        pltpu.async_copy(h.at[srcv.at[0]], rows0, sem0)

        def _pair(t, c2):
            j0 = 2 * t
            j1 = j0 + 1
            pltpu.async_copy(h.at[srcv.at[j1]], rows1, sem1)
            pltpu.make_async_copy(h.at[srcv.at[j0]], rows0, sem0).wait()
            _scale(rows0, j0)
            pltpu.sync_copy(rows0, acc_sh.at[dstv.at[j0]], add=True)

            j2 = jnp.minimum(j0 + 2, QC - 2)
            pltpu.async_copy(h.at[srcv.at[j2]], rows0, sem0)
            pltpu.make_async_copy(h.at[srcv.at[j1]], rows1, sem1).wait()
            _scale(rows1, j1)
            pltpu.sync_copy(rows1, acc_sh.at[dstv.at[j1]], add=True)
            return c2
        lax.fori_loop(0, QC // 2, _pair, None)
        pltpu.make_async_copy(h.at[srcv.at[0]], rows0, sem0).wait()
        return carry
    lax.fori_loop(0, CH // QC, _quarter, None)
    plsc.subcore_barrier()

    pltpu.sync_copy(acc_sh.at[pl.ds(s * RPT, RPT), :],
                    acc_out.at[c, pl.ds(s * RPT, RPT), :])


def _prop(h, srcs, dsts, coefs):
    f = pl.kernel(
        _prop_body,
        out_type=jax.ShapeDtypeStruct((NC, NP, D), _f32),
        mesh=_mesh(),
        compiler_params=pltpu.CompilerParams(needs_layout_passes=False),
        scratch_types=[
            pltpu.VMEM((QC, C), _i32),
            pltpu.VMEM((QC, C), _i32),
            pltpu.VMEM((QC, C), _f32),
            pltpu.VMEM((C, D), _f32),
            pltpu.VMEM((C, D), _f32),
            pltpu.VMEM_SHARED((NP, D), _f32),
            pltpu.SemaphoreType.DMA,
            pltpu.SemaphoreType.DMA,
        ],
    )
    return f(h, srcs, dsts, coefs)


_BM = 512


def _mm0_body(x_ref, w_ref, o_ref):
    o_ref[...] = jnp.dot(x_ref[...], w_ref[...], preferred_element_type=_f32)


def _mm0(xp, W):
    return pl.pallas_call(
        _mm0_body,
        grid=(NP // _BM,),
        in_specs=[pl.BlockSpec((_BM, D), lambda i: (i, 0)),
                  pl.BlockSpec((D, D), lambda i: (0, 0))],
        out_specs=pl.BlockSpec((_BM, D), lambda i: (i, 0)),
        out_shape=jax.ShapeDtypeStruct((NP, D), _f32),
    )(xp, W)


def _mid_body(a0_ref, a1_ref, h_ref, di_ref, b_ref, w_ref, o_ref):
    d = di_ref[...]
    xb = a0_ref[...] + a1_ref[...] + d * d * h_ref[...] + b_ref[...]
    xb = jnp.maximum(xb, 0.0)
    o_ref[...] = jnp.dot(xb, w_ref[...], preferred_element_type=_f32)


def _mid(a0, a1, h, dinv2d, br, W):
    return pl.pallas_call(
        _mid_body,
        grid=(NP // _BM,),
        in_specs=[pl.BlockSpec((_BM, D), lambda i: (i, 0)),
                  pl.BlockSpec((_BM, D), lambda i: (i, 0)),
                  pl.BlockSpec((_BM, D), lambda i: (i, 0)),
                  pl.BlockSpec((_BM, 1), lambda i: (i, 0)),
                  pl.BlockSpec((1, D), lambda i: (0, 0)),
                  pl.BlockSpec((D, D), lambda i: (0, 0))],
        out_specs=pl.BlockSpec((_BM, D), lambda i: (i, 0)),
        out_shape=jax.ShapeDtypeStruct((NP, D), _f32),
    )(a0, a1, h, dinv2d, br, W)


_BM7 = 256


def _read_body(a0_ref, a1_ref, h_ref, di_ref, b_ref, r0_ref, rb0_ref, r1_ref,
               rb1_ref, bt_ref, o_ref, acc_s, acc_c):
    i = pl.program_id(0)

    @pl.when(i == 0)
    def _():
        acc_s[...] = jnp.zeros_like(acc_s)
        acc_c[...] = jnp.zeros_like(acc_c)

    d = di_ref[...]
    x2 = a0_ref[...] + a1_ref[...] + d * d * h_ref[...] + b_ref[...]
    t = jnp.maximum(
        jnp.dot(x2, r0_ref[...], preferred_element_type=_f32) + rb0_ref[...], 0.0)
    r = jnp.dot(t, r1_ref[...], preferred_element_type=_f32) + rb1_ref[0, 0]
    oh = (bt_ref[...] == lax.broadcasted_iota(_i32, (_BM7, 128), 1)).astype(_f32)
    acc_s[...] += lax.dot_general(oh, r, (((0,), (0,)), ((), ())),
                                  preferred_element_type=_f32)
    acc_c[...] += lax.dot_general(oh, jnp.ones((_BM7, 1), _f32),
                                  (((0,), (0,)), ((), ())),
                                  preferred_element_type=_f32)

    @pl.when(i == pl.num_programs(0) - 1)
    def _():
        o_ref[...] = acc_s[...] / jnp.maximum(acc_c[...], 1.0)


def _read(a0, a1, h, dinv2d, br, R0p, rb0p, R1p, rb1p, bt):
    return pl.pallas_call(
        _read_body,
        grid=(NP // _BM7,),
        in_specs=[pl.BlockSpec((_BM7, D), lambda i: (i, 0)),
                  pl.BlockSpec((_BM7, D), lambda i: (i, 0)),
                  pl.BlockSpec((_BM7, D), lambda i: (i, 0)),
                  pl.BlockSpec((_BM7, 1), lambda i: (i, 0)),
                  pl.BlockSpec((1, D), lambda i: (0, 0)),
                  pl.BlockSpec((D, D), lambda i: (0, 0)),
                  pl.BlockSpec((1, D), lambda i: (0, 0)),
                  pl.BlockSpec((D, 1), lambda i: (0, 0)),
                  pl.BlockSpec((1, 1), lambda i: (0, 0)),
                  pl.BlockSpec((_BM7, 1), lambda i: (i, 0))],
        out_specs=pl.BlockSpec((128, 1), lambda i: (0, 0)),
        out_shape=jax.ShapeDtypeStruct((128, 1), _f32),
        scratch_shapes=[pltpu.VMEM((128, 1), _f32),
                        pltpu.VMEM((128, 1), _f32)],
    )(a0, a1, h, dinv2d, br, R0p, rb0p, R1p, rb1p, bt)


def kernel(x, edge_index, edge_attr, batch, W0, b0, W1, b1, W2, b2, R0, rb0, R1, rb1):
    src = edge_index[0]
    dst = edge_index[1]
    srcs = jnp.pad(src, (0, EP - E)).reshape(NW, CH, C)
    dsts = jnp.pad(dst, (0, EP - E)).reshape(NW, CH, C)
    ews = jnp.pad(edge_attr, (0, EP - E)).reshape(NW, CH, C)
    xp = jnp.pad(x, ((0, NP - N), (0, 0)))
    bt = jnp.pad(batch, (0, NP - N), constant_values=127).reshape(NP, 1)
    R0p = jnp.pad(R0, ((0, 0), (0, 128 - R0.shape[1])))
    rb0p = jnp.pad(rb0, (0, 128 - rb0.shape[0])).reshape(1, 128)
    R1p = jnp.pad(R1, ((0, 128 - R1.shape[0]), (0, 0)))
    rb1p = rb1.reshape(1, 1)
    b0r = b0.reshape(1, D)
    b1r = b1.reshape(1, D)
    b2r = b2.reshape(1, D)

    dinv, coef = _pre(dsts, ews, srcs)
    dinv2d = dinv.reshape(NP, 1)

    h0 = _mm0(xp, W0)
    acc = _prop(h0, srcs, dsts, coef)
    h1 = _mid(acc[0], acc[1], h0, dinv2d, b0r, W1)
    acc = _prop(h1, srcs, dsts, coef)
    h2 = _mid(acc[0], acc[1], h1, dinv2d, b1r, W2)
    acc = _prop(h2, srcs, dsts, coef)
    outp = _read(acc[0], acc[1], h2, dinv2d, b2r, R0p, rb0p, R1p, rb1p, bt)
    return outp[:G]
